# Initial kernel scaffold; baseline (speedup 1.0000x reference)
import jax
import jax.numpy as jnp
from jax.experimental import pallas as pl


def kernel(entity_emb, user_emb, entity_2nd_emb, ent_weight_emb, edge_index, edge_type, interact_mat, weight, unmask):
    raise NotImplementedError("write your pallas kernel here")


# speedup vs baseline: 2.7464x; 2.7464x over previous
import functools

import jax
import jax.numpy as jnp
from jax import lax
from jax.experimental import pallas as pl
from jax.experimental.pallas import tpu as pltpu
from jax.experimental.pallas import tpu_sc as plsc

N = 50000
U = 1024
E = 800000
R = 32
D = 64

NC = 2
NS = 16
L = 16
DH = D // NC

CHUNK = 128
NCH = 392
E_PAD = NS * CHUNK * NCH
NROWS = 51200
STRIPE = NROWS // NS
CPS = 3128

KB = 1024
NB = 49


@functools.partial(
    pl.kernel,
    out_type=jax.ShapeDtypeStruct((NC * N, DH), jnp.float32),
    mesh=plsc.VectorSubcoreMesh(core_axis_name="c", subcore_axis_name="s"),
    compiler_params=pltpu.CompilerParams(use_tc_tiling_on_sc=False),
    scratch_types=[
        pltpu.VMEM_SHARED((NROWS, DH), jnp.float32),
        pltpu.VMEM((R * DH,), jnp.float32),
        pltpu.VMEM((CHUNK,), jnp.int32),
        pltpu.VMEM((CHUNK,), jnp.int32),
        pltpu.VMEM((CHUNK,), jnp.int32),
        pltpu.VMEM((CHUNK,), jnp.int32),
        pltpu.VMEM((CHUNK,), jnp.float32),
        pltpu.VMEM((CHUNK, DH), jnp.float32),
        pltpu.SemaphoreType.DMA,
    ],
)
def _sc_entity_agg(head, tail, rel, unmask, emb_stk, weight_h, out,
                   acc, weight_v, sidx_v, tail_v, tidx_v, rel_v, unmask_v,
                   rows_v, sem):
    c = lax.axis_index("c")
    s = lax.axis_index("s")
    cN = c * N

    def _zrow(i, carry):
        for j in range(DH // L):
            rows_v[i, pl.ds(j * L, L)] = jnp.zeros((L,), jnp.float32)
        return carry
    lax.fori_loop(0, CHUNK, _zrow, 0)

    def _zcp(k, carry):
        pltpu.sync_copy(rows_v, acc.at[pl.ds(s * STRIPE + k * CHUNK, CHUNK)])
        return carry
    lax.fori_loop(0, STRIPE // CHUNK, _zcp, 0)

    pltpu.sync_copy(weight_h.at[pl.ds(c * (R * DH), R * DH)], weight_v)
    plsc.subcore_barrier()

    tbase = s * (NCH * CHUNK)

    def _chunk(ci, carry):
        off = tbase + ci * CHUNK
        pltpu.sync_copy(head.at[pl.ds(off, CHUNK)], sidx_v)
        pltpu.sync_copy(tail.at[pl.ds(off, CHUNK)], tail_v)
        pltpu.sync_copy(rel.at[pl.ds(off, CHUNK)], rel_v)
        pltpu.sync_copy(unmask.at[pl.ds(off, CHUNK)], unmask_v)

        def _mkidx(g, inner):
            gb = g * L
            tidx_v[pl.ds(gb, L)] = tail_v[pl.ds(gb, L)] + cN
            return inner
        lax.fori_loop(0, CHUNK // L, _mkidx, 0)

        pltpu.async_copy(emb_stk.at[tidx_v], rows_v, sem).wait()

        def _grp(g, inner):
            gb = g * L
            u_grp = unmask_v[pl.ds(gb, L)]
            r_grp = rel_v[pl.ds(gb, L)]
            for jl in range(L):
                u = u_grp[jl]
                wb = ((r_grp[jl] + (R - 1)) & (R - 1)) * DH
                e = gb + jl
                for j in range(DH // L):
                    v = rows_v[e, pl.ds(j * L, L)]
                    w = weight_v[pl.ds(wb + j * L, L)]
                    rows_v[e, pl.ds(j * L, L)] = v * w * u
            return inner
        lax.fori_loop(0, CHUNK // L, _grp, 0)

        pltpu.sync_copy(rows_v, acc.at[sidx_v], add=True)
        return carry
    lax.fori_loop(0, NCH, _chunk, 0)

    plsc.subcore_barrier()

    roff = jnp.minimum(s * CPS, N - CPS)
    pltpu.sync_copy(acc.at[pl.ds(roff, CPS)], out.at[pl.ds(cN + roff, CPS)])


def _mm_body(a_ref, b_ref, o_ref):
    k = pl.program_id(0)
    rowi = lax.broadcasted_iota(jnp.int32, (KB, D), 0)
    b = jnp.where(k * KB + rowi < N, b_ref[...], 0.0)
    p = jnp.dot(a_ref[...], b, preferred_element_type=jnp.float32)

    @pl.when(k == 0)
    def _init():
        o_ref[...] = p

    @pl.when(k != 0)
    def _acc():
        o_ref[...] = o_ref[...] + p


def _user_agg(interact_mat, entity_emb):
    return pl.pallas_call(
        _mm_body,
        grid=(NB,),
        in_specs=[
            pl.BlockSpec((U, KB), lambda k: (0, k)),
            pl.BlockSpec((KB, D), lambda k: (k, 0)),
        ],
        out_specs=pl.BlockSpec((U, D), lambda k: (0, 0)),
        out_shape=jax.ShapeDtypeStruct((U, D), jnp.float32),
    )(interact_mat, entity_emb)


def kernel(entity_emb, user_emb, entity_2nd_emb, ent_weight_emb,
           edge_index, edge_type, interact_mat, weight, unmask):
    pad = E_PAD - E
    head = jnp.concatenate(
        [edge_index[0].astype(jnp.int32),
         N + (jnp.arange(pad, dtype=jnp.int32) % 1024)])
    tail = jnp.concatenate(
        [edge_index[1].astype(jnp.int32), jnp.zeros((pad,), jnp.int32)])
    rel = jnp.concatenate(
        [edge_type.astype(jnp.int32), jnp.zeros((pad,), jnp.int32)])
    unmask_p = jnp.concatenate(
        [unmask.astype(jnp.float32), jnp.zeros((pad,), jnp.float32)])

    emb_stk = jnp.concatenate([entity_emb[:, :DH], entity_emb[:, DH:]], axis=0)
    weight_h = jnp.concatenate(
        [weight[:, :DH].reshape(-1), weight[:, DH:].reshape(-1)])

    out = _sc_entity_agg(head, tail, rel, unmask_p, emb_stk, weight_h)
    entity_agg = jnp.concatenate([out[:N], out[N:]], axis=1)
    user_agg = _user_agg(interact_mat, entity_emb)
    return (entity_agg, user_agg)


# speedup vs baseline: 4.5921x; 1.6720x over previous
import functools

import jax
import jax.numpy as jnp
from jax import lax
from jax.experimental import pallas as pl
from jax.experimental.pallas import tpu as pltpu
from jax.experimental.pallas import tpu_sc as plsc

N = 50000
U = 1024
E = 800000
R = 32
D = 64

NC = 2
NS = 16
L = 16
DH = D // NC

CHUNK = 128
NCH = 392
NB = NCH // 2
E_PAD = NS * CHUNK * NCH
NROWS = 51200
STRIPE = NROWS // NS
CPS = 3128

KB = 1024
NKB = 49


@functools.partial(
    pl.kernel,
    out_type=jax.ShapeDtypeStruct((NC * N, DH), jnp.float32),
    mesh=plsc.VectorSubcoreMesh(core_axis_name="c", subcore_axis_name="s"),
    compiler_params=pltpu.CompilerParams(use_tc_tiling_on_sc=False),
    scratch_types=[
        pltpu.VMEM_SHARED((NROWS, DH), jnp.float32),
        pltpu.VMEM((8, CHUNK), jnp.int32),
        pltpu.VMEM((8, CHUNK), jnp.int32),
        pltpu.VMEM((CHUNK, DH), jnp.float32),
        pltpu.VMEM((CHUNK, DH), jnp.float32),
        pltpu.VMEM((CHUNK, DH), jnp.float32),
        pltpu.VMEM((CHUNK, DH), jnp.float32),
        pltpu.VMEM((R * DH,), jnp.float32),
        pltpu.SemaphoreType.DMA,
        pltpu.SemaphoreType.DMA,
        pltpu.SemaphoreType.DMA,
        pltpu.SemaphoreType.DMA,
    ],
)
def _sc_entity_agg(idx_pk, emb_stk, weight_h, out,
                   acc, b0, b1, ra0, rb0, ra1, rb1, weight_v,
                   si0, si1, sg0, sg1):
    c = lax.axis_index("c")
    s = lax.axis_index("s")
    cN = c * N
    base = (c * NS + s) * (NB * 8)

    def _zrow(i, carry):
        for j in range(DH // L):
            ra0[i, pl.ds(j * L, L)] = jnp.zeros((L,), jnp.float32)
        return carry
    lax.fori_loop(0, CHUNK, _zrow, 0)

    def _zcp(k, carry):
        pltpu.sync_copy(ra0, acc.at[pl.ds(s * STRIPE + k * CHUNK, CHUNK)])
        return carry
    lax.fori_loop(0, STRIPE // CHUNK, _zcp, 0)

    pltpu.sync_copy(weight_h.at[pl.ds(c * (R * DH), R * DH)], weight_v)
    plsc.subcore_barrier()

    def idx_src(k):
        return idx_pk.at[pl.ds(base + k * 8, 8)]

    def issue_gathers(bb, ra, rb, sg):
        pltpu.async_copy(emb_stk.at[bb.at[1]], ra, sg)
        pltpu.async_copy(emb_stk.at[bb.at[5]], rb, sg)

    def drain_gathers(bb, ra, rb, sg):
        pltpu.make_async_copy(emb_stk.at[bb.at[1]], ra, sg).wait()
        pltpu.make_async_copy(emb_stk.at[bb.at[5]], rb, sg).wait()

    def process_block(bb, ra, rb):
        def g_body(g, carry):
            gb = g * L
            ua_g = lax.bitcast_convert_type(bb[3, pl.ds(gb, L)], jnp.float32)
            ub_g = lax.bitcast_convert_type(bb[7, pl.ds(gb, L)], jnp.float32)
            wa_g = bb[2, pl.ds(gb, L)]
            wb_g = bb[6, pl.ds(gb, L)]
            for jl in range(L):
                e = gb + jl
                ua = ua_g[jl]
                ub = ub_g[jl]
                wa = wa_g[jl]
                wb = wb_g[jl]
                for j in range(DH // L):
                    sl = pl.ds(j * L, L)
                    ra[e, sl] = ra[e, sl] * weight_v[pl.ds(wa + j * L, L)] * ua
                    rb[e, sl] = rb[e, sl] * weight_v[pl.ds(wb + j * L, L)] * ub
            return carry
        lax.fori_loop(0, CHUNK // L, g_body, 0)
        pltpu.sync_copy(ra, acc.at[bb.at[0]], add=True)
        pltpu.sync_copy(rb, acc.at[bb.at[4]], add=True)

    pltpu.async_copy(idx_src(0), b0, si0)
    pltpu.make_async_copy(idx_src(0), b0, si0).wait()
    issue_gathers(b0, ra0, rb0, sg0)
    pltpu.async_copy(idx_src(1), b1, si1)

    def loop_body(jj, carry):
        k0 = 2 * jj
        pltpu.make_async_copy(idx_src(k0 + 1), b1, si1).wait()
        issue_gathers(b1, ra1, rb1, sg1)
        drain_gathers(b0, ra0, rb0, sg0)
        process_block(b0, ra0, rb0)

        @pl.when(jj < NB // 2 - 1)
        def _pf0():
            pltpu.async_copy(idx_src(k0 + 2), b0, si0)

        @pl.when(jj < NB // 2 - 1)
        def _nx1():
            pltpu.make_async_copy(idx_src(k0 + 2), b0, si0).wait()
            issue_gathers(b0, ra0, rb0, sg0)
        drain_gathers(b1, ra1, rb1, sg1)
        process_block(b1, ra1, rb1)

        @pl.when(jj < NB // 2 - 1)
        def _pf1():
            pltpu.async_copy(idx_src(k0 + 3), b1, si1)
        return carry

    lax.fori_loop(0, NB // 2, loop_body, 0)
    plsc.subcore_barrier()

    roff = jnp.minimum(s * CPS, N - CPS)
    pltpu.sync_copy(acc.at[pl.ds(roff, CPS)], out.at[pl.ds(cN + roff, CPS)])


def _mm_body(a_ref, b_ref, o_ref):
    k = pl.program_id(0)
    rowi = lax.broadcasted_iota(jnp.int32, (KB, D), 0)
    b = jnp.where(k * KB + rowi < N, b_ref[...], 0.0)
    p = jnp.dot(a_ref[...], b, preferred_element_type=jnp.float32)

    @pl.when(k == 0)
    def _init():
        o_ref[...] = p

    @pl.when(k != 0)
    def _acc():
        o_ref[...] = o_ref[...] + p


def _user_agg(interact_mat, entity_emb):
    return pl.pallas_call(
        _mm_body,
        grid=(NKB,),
        in_specs=[
            pl.BlockSpec((U, KB), lambda k: (0, k)),
            pl.BlockSpec((KB, D), lambda k: (k, 0)),
        ],
        out_specs=pl.BlockSpec((U, D), lambda k: (0, 0)),
        out_shape=jax.ShapeDtypeStruct((U, D), jnp.float32),
    )(interact_mat, entity_emb)


def kernel(entity_emb, user_emb, entity_2nd_emb, ent_weight_emb,
           edge_index, edge_type, interact_mat, weight, unmask):
    pad = E_PAD - E
    head = jnp.concatenate(
        [edge_index[0].astype(jnp.int32),
         N + (jnp.arange(pad, dtype=jnp.int32) % 1024)])
    tail = jnp.concatenate(
        [edge_index[1].astype(jnp.int32), jnp.zeros((pad,), jnp.int32)])
    woff = jnp.concatenate(
        [((edge_type.astype(jnp.int32) + (R - 1)) & (R - 1)) * DH,
         jnp.zeros((pad,), jnp.int32)])
    ubits = lax.bitcast_convert_type(
        jnp.concatenate([unmask.astype(jnp.float32),
                         jnp.zeros((pad,), jnp.float32)]), jnp.int32)

    halves = []
    for c in range(NC):
        h4 = jnp.stack([head, tail + c * N, woff, ubits])
        a = h4.reshape(4, NS, NB, 2, CHUNK).transpose(1, 2, 3, 0, 4)
        halves.append(a.reshape(NS * NB * 8, CHUNK))
    idx_pk = jnp.concatenate(halves, axis=0)

    emb_stk = jnp.concatenate([entity_emb[:, :DH], entity_emb[:, DH:]], axis=0)
    weight_h = jnp.concatenate(
        [weight[:, :DH].reshape(-1), weight[:, DH:].reshape(-1)])

    out = _sc_entity_agg(idx_pk, emb_stk, weight_h)
    entity_agg = jnp.concatenate([out[:N], out[N:]], axis=1)
    user_agg = _user_agg(interact_mat, entity_emb)
    return (entity_agg, user_agg)
